# baseline (speedup 1.0000x reference)
import dataclasses

import jax
import jax.numpy as jnp
from jax import lax
from jax.experimental import pallas as pl
from jax.experimental.pallas import tpu as pltpu
from jax.experimental.pallas import tpu_sc as plsc

N = 10000
E = 160000
CIN = 256
HID = 512
COUT = 256
HALF = 128
NC = 2
NS = 16
NW = NC * NS
L = 16

EPW = E // NW
DEG_FULL = EPW // L
DEG_REM = EPW - DEG_FULL * L

EPT = E // NS
CW = 80
NCH = EPT // CW
NBUF = 3
NGRP = (NCH + NBUF - 1) // NBUF
RPC = 80
NRC = N // RPC

ROWS = 400
GRID = N // ROWS

_sc_mesh = plsc.VectorSubcoreMesh(
    core_axis_name="c", subcore_axis_name="s", num_cores=NC, num_subcores=NS
)

_sc_params = pltpu.CompilerParams()
if "needs_layout_passes" in pltpu.CompilerParams.__dataclass_fields__:
    _sc_params = dataclasses.replace(_sc_params, needs_layout_passes=False)


def _deg_body(dst_hbm, out_hbm, idx_buf, deg_buf):
    c = lax.axis_index("c")
    s = lax.axis_index("s")
    wid = s * NC + c

    zero16 = jnp.zeros((L,), jnp.float32)
    one16 = jnp.ones((L,), jnp.float32)

    def zero_step(j, carry):
        deg_buf[pl.ds(j * L, L)] = zero16
        return carry

    lax.fori_loop(0, N // L, zero_step, 0)

    pltpu.sync_copy(dst_hbm.at[pl.ds(wid * EPW, EPW)], idx_buf.at[pl.ds(0, EPW)])

    def step(i, carry):
        v = idx_buf[pl.ds(i * L, L)]
        plsc.addupdate_scatter(deg_buf, [v], one16)
        return carry

    lax.fori_loop(0, DEG_FULL, step, 0)

    lane = lax.iota(jnp.int32, L)
    v = idx_buf[pl.ds(DEG_FULL * L, L)]
    msk = lane < DEG_REM
    v = jnp.where(msk, v, 0)
    ones = jnp.where(msk, one16, zero16)
    plsc.addupdate_scatter(deg_buf, [v], ones)

    pltpu.sync_copy(deg_buf, out_hbm.at[wid])


_deg_kernel = pl.kernel(
    _deg_body,
    out_type=jax.ShapeDtypeStruct((NW, N), jnp.float32),
    mesh=_sc_mesh,
    scratch_types=[
        pltpu.VMEM((EPW + L,), jnp.int32),
        pltpu.VMEM((N,), jnp.float32),
    ],
    compiler_params=_sc_params,
)


def _agg_body(g_hbm, srcs_hbm, dst_hbm, zeros_hbm, out_hbm,
              acc, sidx, dring, rows, *sems):
    c = lax.axis_index("c")
    s = lax.axis_index("s")
    dsems = sems[:NBUF]
    gsems = sems[NBUF:2 * NBUF]

    def row_chunks(fn):
        def chunk_step(k, carry):
            chunk = k * NS + s

            @pl.when(chunk < NRC)
            def _():
                fn(chunk * RPC)

            return carry

        lax.fori_loop(0, (NRC + NS - 1) // NS, chunk_step, 0)

    def issue(i, b):
        pltpu.async_copy(dst_hbm.at[pl.ds(s * EPT + i * CW, CW)],
                         dring.at[b], dsems[b])
        pltpu.async_copy(g_hbm.at[sidx.at[i]], rows.at[b], gsems[b])

    pltpu.sync_copy(srcs_hbm.at[c, s], sidx)
    for b in range(NBUF):
        issue(b, b)

    row_chunks(lambda r0: pltpu.sync_copy(zeros_hbm, acc.at[pl.ds(r0, RPC)]))
    plsc.subcore_barrier()

    def group(j, carry):
        for b in range(NBUF):
            i = j * NBUF + b

            @pl.when(i < NCH)
            def _():
                pltpu.make_async_copy(dst_hbm.at[pl.ds(s * EPT, CW)],
                                      dring.at[b], dsems[b]).wait()
                pltpu.make_async_copy(g_hbm.at[sidx.at[i]], rows.at[b],
                                      gsems[b]).wait()
                pltpu.sync_copy(rows.at[b], acc.at[dring.at[b]], add=True)
                nxt = i + NBUF

                @pl.when(nxt < NCH)
                def _():
                    issue(nxt, b)

        return carry

    lax.fori_loop(0, NGRP, group, 0)
    plsc.subcore_barrier()

    row_chunks(lambda r0: pltpu.sync_copy(
        acc.at[pl.ds(r0, RPC)], out_hbm.at[pl.ds(c * N + r0, RPC)]))


_agg_kernel = pl.kernel(
    _agg_body,
    out_type=jax.ShapeDtypeStruct((NC * N, HALF), jnp.float32),
    mesh=_sc_mesh,
    scratch_types=[
        pltpu.VMEM_SHARED((N, HALF), jnp.float32),
        pltpu.VMEM((NCH, CW), jnp.int32),
        pltpu.VMEM((NBUF, CW), jnp.int32),
        pltpu.VMEM((NBUF, CW, HALF), jnp.float32),
    ] + [pltpu.SemaphoreType.DMA] * (2 * NBUF),
    compiler_params=_sc_params,
)


def _prep_body(degp_ref, x_ref, dinvb_ref, g0_ref):
    deg = jnp.sum(degp_ref[...], axis=0) + 1.0
    dinv = lax.rsqrt(deg)
    dinvb = jnp.broadcast_to(dinv[:, None], (N, HALF))
    dinvb_ref[...] = dinvb
    g0_ref[0] = x_ref[:, :HALF] * dinvb
    g0_ref[1] = x_ref[:, HALF:] * dinvb


def _mid_body(dinvb_ref, a0_ref, g0_ref, w1_ref, b1_ref, w2_ref, g2_ref):
    dinv = dinvb_ref[...]
    z0 = (a0_ref[0] + g0_ref[0]) * dinv
    z1 = (a0_ref[1] + g0_ref[1]) * dinv
    z = jnp.concatenate([z0, z1], axis=1)
    h1 = jnp.dot(z.astype(jnp.bfloat16), w1_ref[...].astype(jnp.bfloat16),
                 preferred_element_type=jnp.float32)
    h1 = jnp.maximum(h1 + b1_ref[...][None, :], 0.0)
    h2 = jnp.dot(h1.astype(jnp.bfloat16), w2_ref[...].astype(jnp.bfloat16),
                 preferred_element_type=jnp.float32)
    g2_ref[0] = h2[:, :HALF] * dinv
    g2_ref[1] = h2[:, HALF:] * dinv


def _epi_body(dinvb_ref, a2_ref, g2_ref, b2_ref, out_ref):
    dinv = dinvb_ref[...]
    b2 = b2_ref[...]
    out_ref[:, :HALF] = (a2_ref[0] + g2_ref[0]) * dinv + b2[:HALF][None, :]
    out_ref[:, HALF:] = (a2_ref[1] + g2_ref[1]) * dinv + b2[HALF:][None, :]


_dinvb_spec = pl.BlockSpec((ROWS, HALF), lambda i: (i, 0))
_half_spec = pl.BlockSpec((2, ROWS, HALF), lambda i: (0, i, 0))

_prep_call = pl.pallas_call(
    _prep_body,
    grid=(1,),
    in_specs=[
        pl.BlockSpec((NW, N), lambda i: (0, 0)),
        pl.BlockSpec((N, CIN), lambda i: (0, 0)),
    ],
    out_specs=[
        pl.BlockSpec((N, HALF), lambda i: (0, 0)),
        pl.BlockSpec((2, N, HALF), lambda i: (0, 0, 0)),
    ],
    out_shape=[
        jax.ShapeDtypeStruct((N, HALF), jnp.float32),
        jax.ShapeDtypeStruct((2, N, HALF), jnp.float32),
    ],
)

_mid_call = pl.pallas_call(
    _mid_body,
    grid=(GRID,),
    in_specs=[
        _dinvb_spec,
        _half_spec,
        _half_spec,
        pl.BlockSpec((CIN, HID), lambda i: (0, 0)),
        pl.BlockSpec((HID,), lambda i: (0,)),
        pl.BlockSpec((HID, COUT), lambda i: (0, 0)),
    ],
    out_specs=_half_spec,
    out_shape=jax.ShapeDtypeStruct((2, N, HALF), jnp.float32),
)

_epi_call = pl.pallas_call(
    _epi_body,
    grid=(GRID,),
    in_specs=[
        _dinvb_spec,
        _half_spec,
        _half_spec,
        pl.BlockSpec((COUT,), lambda i: (0,)),
    ],
    out_specs=pl.BlockSpec((ROWS, COUT), lambda i: (i, 0)),
    out_shape=jax.ShapeDtypeStruct((N, COUT), jnp.float32),
)


def kernel(x, edge_index, W1, b1, W2, b2):
    src = edge_index[0].astype(jnp.int32)
    dst = edge_index[1].astype(jnp.int32)
    srcs = jnp.stack([src, src + N]).reshape(NC, NS, NCH, CW)
    zeros = jnp.zeros((RPC, HALF), jnp.float32)

    degp = _deg_kernel(dst)
    dinvb, g0 = _prep_call(degp, x)
    a0 = _agg_kernel(g0.reshape(NC * N, HALF), srcs, dst, zeros)
    g2 = _mid_call(dinvb, a0.reshape(2, N, HALF), g0, W1, b1, W2)
    a2 = _agg_kernel(g2.reshape(NC * N, HALF), srcs, dst, zeros)
    return _epi_call(dinvb, a2.reshape(2, N, HALF), g2, b2)


# speedup vs baseline: 1.1194x; 1.1194x over previous
import dataclasses

import jax
import jax.numpy as jnp
from jax import lax
from jax.experimental import pallas as pl
from jax.experimental.pallas import tpu as pltpu
from jax.experimental.pallas import tpu_sc as plsc

N = 10000
E = 160000
CIN = 256
HID = 512
COUT = 256
HALF = 128
NC = 2
NS = 16
NW = NC * NS
L = 16

EPW = E // NW
DEG_FULL = EPW // L
DEG_REM = EPW - DEG_FULL * L

EPT = E // NS
CW = 80
NCH = EPT // CW
NBUF = 3
NGRP = (NCH + NBUF - 1) // NBUF
RPC = 80
NRC = N // RPC

ROWS = 400
GRID = N // ROWS

_sc_mesh = plsc.VectorSubcoreMesh(
    core_axis_name="c", subcore_axis_name="s", num_cores=NC, num_subcores=NS
)

_sc_params = pltpu.CompilerParams()
if "needs_layout_passes" in pltpu.CompilerParams.__dataclass_fields__:
    _sc_params = dataclasses.replace(_sc_params, needs_layout_passes=False)


def _deg_body(dst_hbm, out_hbm, idx_buf, deg_buf):
    c = lax.axis_index("c")
    s = lax.axis_index("s")
    wid = s * NC + c

    zero16 = jnp.zeros((L,), jnp.float32)
    one16 = jnp.ones((L,), jnp.float32)

    def zero_step(j, carry):
        deg_buf[pl.ds(j * L, L)] = zero16
        return carry

    lax.fori_loop(0, N // L, zero_step, 0)

    pltpu.sync_copy(dst_hbm.at[pl.ds(wid * EPW, EPW)], idx_buf.at[pl.ds(0, EPW)])

    def step(i, carry):
        v = idx_buf[pl.ds(i * L, L)]
        plsc.addupdate_scatter(deg_buf, [v], one16)
        return carry

    lax.fori_loop(0, DEG_FULL, step, 0)

    lane = lax.iota(jnp.int32, L)
    v = idx_buf[pl.ds(DEG_FULL * L, L)]
    msk = lane < DEG_REM
    v = jnp.where(msk, v, 0)
    ones = jnp.where(msk, one16, zero16)
    plsc.addupdate_scatter(deg_buf, [v], ones)

    pltpu.sync_copy(deg_buf, out_hbm.at[wid])


_deg_kernel = pl.kernel(
    _deg_body,
    out_type=jax.ShapeDtypeStruct((NW, N), jnp.float32),
    mesh=_sc_mesh,
    scratch_types=[
        pltpu.VMEM((EPW + L,), jnp.int32),
        pltpu.VMEM((N,), jnp.float32),
    ],
    compiler_params=_sc_params,
)


def _agg_body(g_hbm, srcs_hbm, dst_hbm, out_hbm,
              acc, sidx, dring, rows, *sems):
    c = lax.axis_index("c")
    s = lax.axis_index("s")
    dsems = sems[:NBUF]
    gsems = sems[NBUF:2 * NBUF]

    def row_chunks(fn):
        def chunk_step(k, carry):
            chunk = k * NS + s

            @pl.when(chunk < NRC)
            def _():
                fn(chunk * RPC)

            return carry

        lax.fori_loop(0, (NRC + NS - 1) // NS, chunk_step, 0)

    def issue(i, b):
        pltpu.async_copy(dst_hbm.at[pl.ds(s * EPT + i * CW, CW)],
                         dring.at[b], dsems[b])
        pltpu.async_copy(g_hbm.at[sidx.at[i]], rows.at[b], gsems[b])

    pltpu.sync_copy(srcs_hbm.at[c, s], sidx)
    for b in range(NBUF - 1):
        issue(b, b)

    zb = NBUF - 1
    zero16 = jnp.zeros((L,), jnp.float32)

    def zero_row(r, carry):
        for j in range(HALF // L):
            rows[zb, r, pl.ds(j * L, L)] = zero16
        return carry

    lax.fori_loop(0, CW, zero_row, 0)
    row_chunks(lambda r0: pltpu.sync_copy(rows.at[zb], acc.at[pl.ds(r0, RPC)]))
    plsc.subcore_barrier()
    issue(zb, zb)

    def group(j, carry):
        for b in range(NBUF):
            i = j * NBUF + b

            @pl.when(i < NCH)
            def _():
                pltpu.make_async_copy(dst_hbm.at[pl.ds(s * EPT, CW)],
                                      dring.at[b], dsems[b]).wait()
                pltpu.make_async_copy(g_hbm.at[sidx.at[i]], rows.at[b],
                                      gsems[b]).wait()
                pltpu.sync_copy(rows.at[b], acc.at[dring.at[b]], add=True)
                nxt = i + NBUF

                @pl.when(nxt < NCH)
                def _():
                    issue(nxt, b)

        return carry

    lax.fori_loop(0, NGRP, group, 0)
    plsc.subcore_barrier()

    row_chunks(lambda r0: pltpu.sync_copy(
        acc.at[pl.ds(r0, RPC)], out_hbm.at[pl.ds(c * N + r0, RPC)]))


_agg_kernel = pl.kernel(
    _agg_body,
    out_type=jax.ShapeDtypeStruct((NC * N, HALF), jnp.float32),
    mesh=_sc_mesh,
    scratch_types=[
        pltpu.VMEM_SHARED((N, HALF), jnp.float32),
        pltpu.VMEM((NCH, CW), jnp.int32),
        pltpu.VMEM((NBUF, CW), jnp.int32),
        pltpu.VMEM((NBUF, CW, HALF), jnp.float32),
    ] + [pltpu.SemaphoreType.DMA] * (2 * NBUF),
    compiler_params=_sc_params,
)


def _prep_body(degp_ref, x_ref, dinvb_ref, g0_ref):
    deg = jnp.sum(degp_ref[...], axis=0) + 1.0
    dinv = lax.rsqrt(deg)
    dinvb = jnp.broadcast_to(dinv[:, None], (N, HALF))
    dinvb_ref[...] = dinvb
    g0_ref[0] = x_ref[:, :HALF] * dinvb
    g0_ref[1] = x_ref[:, HALF:] * dinvb


def _mid_body(dinvb_ref, a0_ref, g0_ref, w1_ref, b1_ref, w2_ref, g2_ref):
    dinv = dinvb_ref[...]
    z0 = (a0_ref[0] + g0_ref[0]) * dinv
    z1 = (a0_ref[1] + g0_ref[1]) * dinv
    z = jnp.concatenate([z0, z1], axis=1)
    h1 = jnp.dot(z.astype(jnp.bfloat16), w1_ref[...].astype(jnp.bfloat16),
                 preferred_element_type=jnp.float32)
    h1 = jnp.maximum(h1 + b1_ref[...][None, :], 0.0)
    h2 = jnp.dot(h1.astype(jnp.bfloat16), w2_ref[...].astype(jnp.bfloat16),
                 preferred_element_type=jnp.float32)
    g2_ref[0] = h2[:, :HALF] * dinv
    g2_ref[1] = h2[:, HALF:] * dinv


def _epi_body(dinvb_ref, a2_ref, g2_ref, b2_ref, out_ref):
    dinv = dinvb_ref[...]
    b2 = b2_ref[...]
    out_ref[:, :HALF] = (a2_ref[0] + g2_ref[0]) * dinv + b2[:HALF][None, :]
    out_ref[:, HALF:] = (a2_ref[1] + g2_ref[1]) * dinv + b2[HALF:][None, :]


_dinvb_spec = pl.BlockSpec((ROWS, HALF), lambda i: (i, 0))
_half_spec = pl.BlockSpec((2, ROWS, HALF), lambda i: (0, i, 0))

_prep_call = pl.pallas_call(
    _prep_body,
    grid=(1,),
    in_specs=[
        pl.BlockSpec((NW, N), lambda i: (0, 0)),
        pl.BlockSpec((N, CIN), lambda i: (0, 0)),
    ],
    out_specs=[
        pl.BlockSpec((N, HALF), lambda i: (0, 0)),
        pl.BlockSpec((2, N, HALF), lambda i: (0, 0, 0)),
    ],
    out_shape=[
        jax.ShapeDtypeStruct((N, HALF), jnp.float32),
        jax.ShapeDtypeStruct((2, N, HALF), jnp.float32),
    ],
)

_mid_call = pl.pallas_call(
    _mid_body,
    grid=(GRID,),
    in_specs=[
        _dinvb_spec,
        _half_spec,
        _half_spec,
        pl.BlockSpec((CIN, HID), lambda i: (0, 0)),
        pl.BlockSpec((HID,), lambda i: (0,)),
        pl.BlockSpec((HID, COUT), lambda i: (0, 0)),
    ],
    out_specs=_half_spec,
    out_shape=jax.ShapeDtypeStruct((2, N, HALF), jnp.float32),
)

_epi_call = pl.pallas_call(
    _epi_body,
    grid=(GRID,),
    in_specs=[
        _dinvb_spec,
        _half_spec,
        _half_spec,
        pl.BlockSpec((COUT,), lambda i: (0,)),
    ],
    out_specs=pl.BlockSpec((ROWS, COUT), lambda i: (i, 0)),
    out_shape=jax.ShapeDtypeStruct((N, COUT), jnp.float32),
)


def kernel(x, edge_index, W1, b1, W2, b2):
    src = edge_index[0].astype(jnp.int32)
    dst = edge_index[1].astype(jnp.int32)
    srcs = jnp.stack([src, src + N]).reshape(NC, NS, NCH, CW)

    degp = _deg_kernel(dst)
    dinvb, g0 = _prep_call(degp, x)
    a0 = _agg_kernel(g0.reshape(NC * N, HALF), srcs, dst)
    g2 = _mid_call(dinvb, a0.reshape(2, N, HALF), g0, W1, b1, W2)
    a2 = _agg_kernel(g2.reshape(NC * N, HALF), srcs, dst)
    return _epi_call(dinvb, a2.reshape(2, N, HALF), g2, b2)
